# baseline (speedup 1.0000x reference)
import functools

import jax
import jax.numpy as jnp
from jax import lax
from jax.experimental import pallas as pl
from jax.experimental.pallas import tpu as pltpu
from jax.experimental.pallas import tpu_sc as plsc

ANUM = 2
N_ETYPES = 8
E_HID = 64
C = 128
K = 16

_PREC = jax.lax.Precision.DEFAULT


def _dot(a, b):
    return jnp.dot(a, b, preferred_element_type=jnp.float32, precision=_PREC)


def _dense_body(pts_ref, knn_ref, We1_ref, be1_ref, We2_ref, be2_ref,
                Wg_ref, bg_ref, Wd_ref, bd_ref, out_ref):
    nb = pts_ref.shape[1]
    ctr = pts_ref[...]
    We1 = We1_ref[...]
    be1 = be1_ref[...]
    We2 = We2_ref[...]
    be2 = be2_ref[...]
    knnTs = []
    logits = []
    for k in range(K):
        pkT = knn_ref[k].T
        knnTs.append(pkT)
        h = jnp.maximum(_dot(We1, pkT - ctr) + be1, 0.0)
        logits.append(_dot(We2, h) + be2)
    L3 = jnp.concatenate(logits, axis=0).reshape(K, N_ETYPES, nb)
    m = jnp.max(L3, axis=1, keepdims=True)
    p = jnp.exp(L3 - m)
    s = jnp.sum(p, axis=1, keepdims=True)
    w3 = p / s
    aggs = []
    for t in range(N_ETYPES):
        acc = w3[0, t][None, :] * knnTs[0]
        for k in range(1, K):
            acc = acc + w3[k, t][None, :] * knnTs[k]
        aggs.append(acc)
    aggf = jnp.concatenate(aggs, axis=0) * (1.0 / K)
    for a in range(ANUM):
        msg = _dot(Wg_ref[a], aggf) + bg_ref[a]
        x = jnp.maximum(ctr + msg, 0.0)
        out_ref[a] = _dot(Wd_ref[a], x) + bd_ref[a]


def _dense_call(pts2, knn_planes, We1, be1c, We2, be2c, Wg, bgc, Wd, bdc, *, nb):
    n = pts2.shape[1]
    hid2 = Wd.shape[1]
    grid = (n // nb,)
    return pl.pallas_call(
        _dense_body,
        grid=grid,
        in_specs=[
            pl.BlockSpec((C, nb), lambda i: (0, i)),
            pl.BlockSpec((K, nb, C), lambda i: (0, i, 0)),
            pl.BlockSpec((E_HID, C), lambda i: (0, 0)),
            pl.BlockSpec((E_HID, 1), lambda i: (0, 0)),
            pl.BlockSpec((N_ETYPES, E_HID), lambda i: (0, 0)),
            pl.BlockSpec((N_ETYPES, 1), lambda i: (0, 0)),
            pl.BlockSpec((ANUM, C, N_ETYPES * C), lambda i: (0, 0, 0)),
            pl.BlockSpec((ANUM, C, 1), lambda i: (0, 0, 0)),
            pl.BlockSpec((ANUM, hid2, C), lambda i: (0, 0, 0)),
            pl.BlockSpec((ANUM, hid2, 1), lambda i: (0, 0, 0)),
        ],
        out_specs=pl.BlockSpec((ANUM, hid2, nb), lambda i: (0, 0, i)),
        out_shape=jax.ShapeDtypeStruct((ANUM, hid2, n), jnp.float32),
    )(pts2, knn_planes, We1, be1c, We2, be2c, Wg, bgc, Wd, bdc)


_NP = 10240
_CHUNK = 128
_NW = 32
_EP = K * _NP
_NCHUNK = _EP // _NW // _CHUNK


def _sc_gather(ptsT, gidx, sidx):
    mesh = plsc.VectorSubcoreMesh(core_axis_name="c", subcore_axis_name="s")

    NBUF = 4

    @functools.partial(
        pl.kernel,
        mesh=mesh,
        out_type=jax.ShapeDtypeStruct((_EP, C), jnp.float32),
        scratch_types=(
            [pltpu.VMEM((_NCHUNK, _CHUNK), jnp.int32)] * 2
            + [pltpu.VMEM((_CHUNK, C), jnp.float32)] * NBUF
            + [pltpu.SemaphoreType.DMA] * (2 * NBUF)
        ),
    )
    def gkern(ptsT_hbm, gidx_hbm, sidx_hbm, out_hbm, gidx_v, sidx_v, *bufs):
        rows = bufs[:NBUF]
        gsem = bufs[NBUF:2 * NBUF]
        wsem = bufs[2 * NBUF:]
        wid = lax.axis_index("s") * 2 + lax.axis_index("c")
        pltpu.sync_copy(gidx_hbm.at[wid], gidx_v)
        pltpu.sync_copy(sidx_hbm.at[wid], sidx_v)

        def gather(b, ch):
            pltpu.async_copy(ptsT_hbm.at[gidx_v.at[ch]], rows[b], gsem[b])

        def gather_wait(b, ch):
            pltpu.make_async_copy(ptsT_hbm.at[gidx_v.at[ch]], rows[b], gsem[b]).wait()

        def write(b, ch):
            pltpu.async_copy(rows[b], out_hbm.at[sidx_v.at[ch]], wsem[b])

        def write_wait(b, ch):
            pltpu.make_async_copy(rows[b], out_hbm.at[sidx_v.at[ch]], wsem[b]).wait()

        for b in range(NBUF):
            gather(b, b)

        def step(j, carry):
            for b in range(NBUF):
                ch = NBUF * j + b
                gather_wait(b, ch)
                write(b, ch)
            for b in range(NBUF):
                ch = NBUF * j + b
                write_wait(b, ch)

                @pl.when(j < _NCHUNK // NBUF - 1)
                def _():
                    gather(b, ch + NBUF)
            return carry

        lax.fori_loop(0, _NCHUNK // NBUF, step, 0)

    return gkern(ptsT, gidx, sidx)


def _tc_transpose(pts2):
    NP = pts2.shape[1]
    nb = 512

    def body(x_ref, o_ref):
        o_ref[...] = x_ref[...].T

    return pl.pallas_call(
        body,
        grid=(NP // nb,),
        in_specs=[pl.BlockSpec((C, nb), lambda i: (0, i))],
        out_specs=pl.BlockSpec((nb, C), lambda i: (i, 0)),
        out_shape=jax.ShapeDtypeStruct((NP, C), jnp.float32),
    )(pts2)


def kernel(pts, nn_idx, We1, be1, We2, be2, Wg, bg, Wd, bd):
    B, c, N = pts.shape
    k = nn_idx.shape[-1]
    NP = _NP
    pts2 = jnp.pad(pts[0], ((0, 0), (0, NP - N)))
    ptsT = _tc_transpose(pts2)
    e = jnp.arange(_EP, dtype=jnp.int32)
    gidx = jnp.pad(nn_idx[0].reshape(N * k), (0, _EP - N * k)).reshape(_NW, _NCHUNK, _CHUNK)
    sidx = ((e & (k - 1)) * NP + (e >> 4)).reshape(_NW, _NCHUNK, _CHUNK)
    knn_planes = _sc_gather(ptsT, gidx, sidx).reshape(k, NP, c)
    out = _dense_call(
        pts2, knn_planes,
        We1, be1[:, None], We2, be2[:, None],
        Wg, bg[:, :, None], Wd, bd[:, :, None],
        nb=1024,
    )
    return out[None, :, :, :N, None]
